# Initial kernel scaffold; baseline (speedup 1.0000x reference)
import jax
import jax.numpy as jnp
from jax.experimental import pallas as pl


def kernel(points, W1, b1, W2, b2, W3, b3):
    raise NotImplementedError("write your pallas kernel here")


# speedup vs baseline: 16.5099x; 16.5099x over previous
import functools

import jax
import jax.numpy as jnp
from jax.experimental import pallas as pl

K_NN = 32
ROW_BLOCK = 256


def _mlp_body(pr_ref, w1_ref, b1_ref, w2_ref, b2_ref, w3_ref, b3_ref, f3_ref):
    p = pr_ref[0]
    f = jnp.maximum(jnp.dot(p, w1_ref[...], preferred_element_type=jnp.float32)
                    + b1_ref[...], 0.0)
    f = jnp.maximum(jnp.dot(f, w2_ref[...], preferred_element_type=jnp.float32)
                    + b2_ref[...], 0.0)
    f = jnp.maximum(jnp.dot(f, w3_ref[...], preferred_element_type=jnp.float32)
                    + b3_ref[...], 0.0)
    f3_ref[0] = f


def _knn_pool_body(pr_ref, pt_ref, f3_ref, out_ref):
    q = pr_ref[0]
    pt = pt_ref[0]

    d2 = None
    for c in range(3):
        diff = q[:, c:c + 1] - pt[c:c + 1, :]
        sq = diff * diff
        d2 = sq if d2 is None else d2 + sq

    bits = jax.lax.bitcast_convert_type(d2, jnp.int32)

    def body(i, t):
        b = 30 - i
        cand = t | jnp.left_shift(jnp.int32(1), b)
        cnt = jnp.sum((bits < cand).astype(jnp.int32), axis=1, keepdims=True)
        return jnp.where(cnt < K_NN, cand, t)

    t0 = jnp.zeros((bits.shape[0], 1), jnp.int32)
    t = jax.lax.fori_loop(0, 31, body, t0)

    m = jnp.sum((bits < t).astype(jnp.int32), axis=1, keepdims=True)
    e = jnp.sum((bits == t).astype(jnp.int32), axis=1, keepdims=True)
    tie_w = (K_NN - m).astype(jnp.float32) / e.astype(jnp.float32)
    w = jnp.where(bits < t, 1.0, jnp.where(bits == t, tie_w, 0.0))

    out_ref[0] = jnp.dot(w, f3_ref[0], preferred_element_type=jnp.float32) \
        * (1.0 / K_NN)


def kernel(points, W1, b1, W2, b2, W3, b3):
    B, N, C = points.shape
    pr = jnp.pad(points, ((0, 0), (0, 0), (0, 8 - C)))
    pt = jnp.swapaxes(pr, 1, 2)
    w1p = jnp.pad(W1, ((0, 8 - C), (0, 0)))

    f3 = pl.pallas_call(
        _mlp_body,
        grid=(B,),
        in_specs=[
            pl.BlockSpec((1, N, 8), lambda b: (b, 0, 0)),
            pl.BlockSpec((8, 64), lambda b: (0, 0)),
            pl.BlockSpec((1, 64), lambda b: (0, 0)),
            pl.BlockSpec((64, 64), lambda b: (0, 0)),
            pl.BlockSpec((1, 64), lambda b: (0, 0)),
            pl.BlockSpec((64, 128), lambda b: (0, 0)),
            pl.BlockSpec((1, 128), lambda b: (0, 0)),
        ],
        out_specs=pl.BlockSpec((1, N, 128), lambda b: (b, 0, 0)),
        out_shape=jax.ShapeDtypeStruct((B, N, 128), jnp.float32),
    )(pr, w1p, b1[None], W2, b2[None], W3, b3[None])

    n_rb = N // ROW_BLOCK
    out = pl.pallas_call(
        _knn_pool_body,
        grid=(B, n_rb),
        in_specs=[
            pl.BlockSpec((1, ROW_BLOCK, 8), lambda b, r: (b, r, 0)),
            pl.BlockSpec((1, 8, N), lambda b, r: (b, 0, 0)),
            pl.BlockSpec((1, N, 128), lambda b, r: (b, 0, 0)),
        ],
        out_specs=pl.BlockSpec((1, ROW_BLOCK, 128), lambda b, r: (b, r, 0)),
        out_shape=jax.ShapeDtypeStruct((B, N, 128), jnp.float32),
    )(pr, pt, f3)
    return out
